# baseline (speedup 1.0000x reference)
import functools

import jax
import jax.numpy as jnp
from jax import lax
from jax.experimental import pallas as pl
from jax.experimental.pallas import tpu as pltpu
from jax.experimental.pallas import tpu_sc as plsc

N = 10000
E = 320000
NC = 2
NS = 16
NW = NC * NS
B = 96
STEPS2 = 212
PAIRS2 = STEPS2 // 2
STEPS = STEPS2 // 2
EP = NS * STEPS2 * B
NA = 10112
RPT = NA // NS
_CHUNKS = [B] * (RPT // B) + ([RPT % B] if RPT % B else [])
_HS = 632

_MESH = plsc.VectorSubcoreMesh(core_axis_name="c", subcore_axis_name="s")
_SC_PARAMS = pltpu.CompilerParams(use_tc_tiling_on_sc=False)


@functools.partial(
    pl.kernel,
    out_type=jax.ShapeDtypeStruct((NC, N), jnp.float32),
    mesh=_MESH,
    scratch_types=[
        pltpu.VMEM((B,), jnp.float32),
        pltpu.VMEM((STEPS, B), jnp.int32),
        pltpu.VMEM((STEPS, B), jnp.int32),
        pltpu.VMEM_SHARED((NA,), jnp.float32),
    ],
    compiler_params=_SC_PARAMS,
)
def _deg(e_hbm, ones_hbm, zeros_hbm, deg_hbm, ones_v, pidx, idx_v, acc):
    c = lax.axis_index("c")
    s = lax.axis_index("s")
    pltpu.sync_copy(ones_hbm, ones_v)

    @pl.when(s == 0)
    def _zero_acc():
        pltpu.sync_copy(zeros_hbm, acc)

    pltpu.sync_copy(e_hbm.at[s, pl.ds(c * STEPS, STEPS)], pidx)

    def urow(i, carry):
        def ucol(k, carry2):
            v = pidx[i, pl.ds(k * 16, 16)]
            idx_v[i, pl.ds(k * 16, 16)] = lax.shift_right_logical(v, 14)
            return carry2
        return lax.fori_loop(0, B // 16, ucol, carry)

    lax.fori_loop(0, STEPS, urow, 0)
    plsc.subcore_barrier()

    def step(i, carry):
        pltpu.sync_copy(ones_v, acc.at[idx_v.at[i]], add=True)
        return carry

    lax.fori_loop(0, STEPS, step, 0)
    plsc.subcore_barrier()

    @pl.when(s == 0)
    def _copy_out():
        pltpu.sync_copy(acc.at[pl.ds(0, N)], deg_hbm.at[c])


def _make_agg(HD):
    @functools.partial(
        pl.kernel,
        out_type=jax.ShapeDtypeStruct((NC, NA, HD), jnp.float32),
        mesh=_MESH,
        scratch_types=[
            pltpu.VMEM((STEPS, B), jnp.int32),
            pltpu.VMEM((STEPS, B), jnp.int32),
            pltpu.VMEM((STEPS, B), jnp.int32),
            pltpu.VMEM((B, HD), jnp.float32),
            pltpu.VMEM((B, HD), jnp.float32),
            pltpu.VMEM_SHARED((N, HD), jnp.float32),
            pltpu.VMEM_SHARED((NA, HD), jnp.float32),
            pltpu.SemaphoreType.DMA,
            pltpu.SemaphoreType.DMA,
            pltpu.SemaphoreType.DMA,
            pltpu.SemaphoreType.DMA,
        ],
        compiler_params=_SC_PARAMS,
    )
    def agg(h_hbm, e_hbm, out_hbm, pidx, sidx, didx, rows0, rows1,
            h_sh, acc, gsem0, gsem1, ssem0, ssem1):
        c = lax.axis_index("c")
        s = lax.axis_index("s")
        zero = jnp.zeros((16,), jnp.float32)

        hbase = s * _HS

        @pl.when(s < NS - 1)
        def _stage():
            pltpu.sync_copy(h_hbm.at[c, pl.ds(hbase, _HS)],
                            h_sh.at[pl.ds(hbase, _HS)])

        @pl.when(s == NS - 1)
        def _stage_last():
            pltpu.sync_copy(h_hbm.at[c, pl.ds(hbase, N - (NS - 1) * _HS)],
                            h_sh.at[pl.ds(hbase, N - (NS - 1) * _HS)])

        def zrow(r, carry):
            def zcol(k, carry2):
                rows0[r, pl.ds(k * 16, 16)] = zero
                return carry2
            return lax.fori_loop(0, HD // 16, zcol, carry)

        lax.fori_loop(0, B, zrow, 0)
        base = s * RPT
        off = 0
        for n in _CHUNKS:
            pltpu.sync_copy(rows0.at[pl.ds(0, n)],
                            acc.at[pl.ds(base + off, n)])
            off += n
        plsc.subcore_barrier()
        mask14 = jnp.full((16,), 16383, jnp.int32)
        npairs = STEPS // 2

        for ph in range(2):
            pltpu.sync_copy(e_hbm.at[s, pl.ds(ph * STEPS, STEPS)], pidx)

            def urow(i, carry):
                def ucol(k, carry2):
                    v = pidx[i, pl.ds(k * 16, 16)]
                    sidx[i, pl.ds(k * 16, 16)] = lax.bitwise_and(v, mask14)
                    didx[i, pl.ds(k * 16, 16)] = (
                        lax.shift_right_logical(v, 14))
                    return carry2
                return lax.fori_loop(0, B // 16, ucol, carry)

            lax.fori_loop(0, STEPS, urow, 0)

            pltpu.async_copy(h_sh.at[sidx.at[0]], rows0, gsem0)
            pltpu.async_copy(h_sh.at[sidx.at[1]], rows1, gsem1)

            def pair(p, carry):
                i0 = 2 * p
                pltpu.make_async_copy(h_sh.at[sidx.at[i0]], rows0,
                                      gsem0).wait()
                pltpu.async_copy(rows0, acc.at[didx.at[i0]], ssem0, add=True)
                pltpu.make_async_copy(h_sh.at[sidx.at[i0 + 1]], rows1,
                                      gsem1).wait()
                pltpu.async_copy(rows1, acc.at[didx.at[i0 + 1]], ssem1,
                                 add=True)

                @pl.when(p < npairs - 1)
                def _prefetch():
                    pltpu.make_async_copy(rows0, acc.at[didx.at[i0]],
                                          ssem0).wait()
                    pltpu.async_copy(h_sh.at[sidx.at[i0 + 2]], rows0, gsem0)
                    pltpu.make_async_copy(rows1, acc.at[didx.at[i0 + 1]],
                                          ssem1).wait()
                    pltpu.async_copy(h_sh.at[sidx.at[i0 + 3]], rows1, gsem1)

                return carry

            lax.fori_loop(0, npairs, pair, 0)
            pltpu.make_async_copy(rows0, acc.at[didx.at[STEPS - 2]],
                                  ssem0).wait()
            pltpu.make_async_copy(rows1, acc.at[didx.at[STEPS - 1]],
                                  ssem1).wait()

        plsc.subcore_barrier()
        off = 0
        for n in _CHUNKS:
            pltpu.sync_copy(acc.at[pl.ds(base + off, n)],
                            out_hbm.at[c, pl.ds(base + off, n)])
            off += n

    return agg


_agg64h = _make_agg(64)
_agg32h = _make_agg(32)


GB = 2000
GRID = N // GB
_DOT = dict(preferred_element_type=jnp.float32, precision=lax.Precision.HIGHEST)


def _split(t):
    hd = t.shape[1] // 2
    return jnp.stack([t[:, :hd], t[:, hd:]])


def _mm_scale_body(deg_ref, x_ref, w_ref, o_ref):
    dinv = lax.rsqrt(1.0 + deg_ref[0] + deg_ref[1])
    o_ref[...] = _split(lax.dot_general(x_ref[...] * dinv, w_ref[...],
                                        (((1,), (0,)), ((), ())), **_DOT))


def _fused_body(agg_ref, h_ref, deg_ref, b_ref, w_ref, o_ref):
    dinv = lax.rsqrt(1.0 + deg_ref[0] + deg_ref[1])
    aggh = jnp.concatenate([agg_ref[0] + h_ref[0], agg_ref[1] + h_ref[1]],
                           axis=1)
    t = jnp.maximum(dinv * aggh + b_ref[...], 0.0)
    o_ref[...] = _split(lax.dot_general(t * dinv, w_ref[...],
                                        (((1,), (0,)), ((), ())), **_DOT))


def _final_body(agg_ref, h_ref, deg_ref, b_ref, o_ref):
    dinv = lax.rsqrt(1.0 + deg_ref[0] + deg_ref[1])
    aggh = jnp.concatenate([agg_ref[0] + h_ref[0], agg_ref[1] + h_ref[1]],
                           axis=1)
    o_ref[...] = dinv * aggh + b_ref[...]


def _row_spec(d):
    return pl.BlockSpec((GB, d), lambda i: (i, 0))


def _half_spec(hd):
    return pl.BlockSpec((NC, GB, hd), lambda i: (0, i, 0))


_DEG_SPEC = pl.BlockSpec((NC, GB, 1), lambda i: (0, i, 0))


def _w_spec(din, dout):
    return pl.BlockSpec((din, dout), lambda i: (0, 0))


def _b_spec(d):
    return pl.BlockSpec((1, d), lambda i: (0, 0))


_mm_scale = pl.pallas_call(
    _mm_scale_body, grid=(GRID,),
    in_specs=[_DEG_SPEC, _row_spec(128), _w_spec(128, 128)],
    out_specs=_half_spec(64),
    out_shape=jax.ShapeDtypeStruct((NC, N, 64), jnp.float32),
)


def _make_fused(dout):
    return pl.pallas_call(
        _fused_body, grid=(GRID,),
        in_specs=[_half_spec(64), _half_spec(64), _DEG_SPEC, _b_spec(128),
                  _w_spec(128, dout)],
        out_specs=_half_spec(dout // 2),
        out_shape=jax.ShapeDtypeStruct((NC, N, dout // 2), jnp.float32),
    )


_fused128 = _make_fused(128)
_fused64 = _make_fused(64)

_final = pl.pallas_call(
    _final_body, grid=(GRID,),
    in_specs=[_half_spec(32), _half_spec(32), _DEG_SPEC, _b_spec(64)],
    out_specs=_row_spec(64),
    out_shape=jax.ShapeDtypeStruct((N, 64), jnp.float32),
)


def kernel(x, edge_index, W1, b1, W2, b2, W3, b3):
    pad = EP - E
    ar = jnp.arange(pad, dtype=jnp.int32)
    pad_src = (ar * 131) % N
    pad_dst = N + ar % (NA - N)
    src2 = jnp.concatenate([edge_index[0].astype(jnp.int32), pad_src])
    dst2 = jnp.concatenate([edge_index[1].astype(jnp.int32), pad_dst])
    e2 = (dst2 * 16384 + src2).reshape(NS, STEPS2, B)
    ones1 = jnp.ones((B,), jnp.float32)
    zeros1 = jnp.zeros((NA,), jnp.float32)
    deg2 = _deg(e2, ones1, zeros1)[:, :, None]
    h1p = _mm_scale(deg2, x, W1)
    agg1 = _agg64h(h1p, e2)
    h2p = _fused128(agg1, h1p, deg2, b1.reshape(1, -1), W2)
    agg2 = _agg64h(h2p, e2)
    h3p = _fused64(agg2, h2p, deg2, b2.reshape(1, -1), W3)
    agg3 = _agg32h(h3p, e2)
    return _final(agg3, h3p, deg2, b3.reshape(1, -1))


# speedup vs baseline: 1.1098x; 1.1098x over previous
import functools

import jax
import jax.numpy as jnp
from jax import lax
from jax.experimental import pallas as pl
from jax.experimental.pallas import tpu as pltpu
from jax.experimental.pallas import tpu_sc as plsc

N = 10000
E = 320000
NC = 2
NS = 16
NW = NC * NS
B = 80
STEPS = 125
PAIRS = (STEPS - 1) // 2
NA = 10112
RPT = NA // NS
_CHUNKS = [B] * (RPT // B) + ([RPT % B] if RPT % B else [])

_MESH = plsc.VectorSubcoreMesh(core_axis_name="c", subcore_axis_name="s")
_SC_PARAMS = pltpu.CompilerParams(use_tc_tiling_on_sc=False)


@functools.partial(
    pl.kernel,
    out_type=jax.ShapeDtypeStruct((NC, N), jnp.float32),
    mesh=_MESH,
    scratch_types=[
        pltpu.VMEM((B,), jnp.float32),
        pltpu.VMEM((STEPS, B), jnp.int32),
        pltpu.VMEM_SHARED((NA,), jnp.float32),
    ],
    compiler_params=_SC_PARAMS,
)
def _deg(e_hbm, ones_hbm, zeros_hbm, deg_hbm, ones_v, idx_v, acc):
    c = lax.axis_index("c")
    s = lax.axis_index("s")
    wid = c * NS + s
    pltpu.sync_copy(ones_hbm, ones_v)

    @pl.when(s == 0)
    def _zero_acc():
        pltpu.sync_copy(zeros_hbm, acc)

    pltpu.sync_copy(e_hbm.at[1, wid], idx_v)
    plsc.subcore_barrier()

    def step(i, carry):
        pltpu.sync_copy(ones_v, acc.at[idx_v.at[i]], add=True)
        return carry

    lax.fori_loop(0, STEPS, step, 0)
    plsc.subcore_barrier()

    @pl.when(s == 0)
    def _copy_out():
        pltpu.sync_copy(acc.at[pl.ds(0, N)], deg_hbm.at[c])


def _make_agg(D):
    @functools.partial(
        pl.kernel,
        out_type=jax.ShapeDtypeStruct((NC, NA, D), jnp.float32),
        mesh=_MESH,
        scratch_types=[
            pltpu.VMEM((STEPS, B), jnp.int32),
            pltpu.VMEM((STEPS, B), jnp.int32),
            pltpu.VMEM((B, D), jnp.float32),
            pltpu.VMEM((B, D), jnp.float32),
            pltpu.VMEM_SHARED((NA, D), jnp.float32),
            pltpu.SemaphoreType.DMA,
            pltpu.SemaphoreType.DMA,
            pltpu.SemaphoreType.DMA,
            pltpu.SemaphoreType.DMA,
        ],
        compiler_params=_SC_PARAMS,
    )
    def agg(h_hbm, e_hbm, out_hbm, sidx, didx, rows0, rows1, acc,
            gsem0, gsem1, ssem0, ssem1):
        c = lax.axis_index("c")
        s = lax.axis_index("s")
        wid = c * NS + s
        zero = jnp.zeros((16,), jnp.float32)

        def zrow(r, carry):
            def zcol(k, carry2):
                rows0[r, pl.ds(k * 16, 16)] = zero
                return carry2
            return lax.fori_loop(0, D // 16, zcol, carry)

        lax.fori_loop(0, B, zrow, 0)
        base = s * RPT
        off = 0
        for n in _CHUNKS:
            pltpu.sync_copy(rows0.at[pl.ds(0, n)],
                            acc.at[pl.ds(base + off, n)])
            off += n
        pltpu.sync_copy(e_hbm.at[0, wid], sidx)
        pltpu.sync_copy(e_hbm.at[1, wid], didx)
        plsc.subcore_barrier()

        pltpu.async_copy(h_hbm.at[sidx.at[0]], rows0, gsem0)
        pltpu.async_copy(h_hbm.at[sidx.at[1]], rows1, gsem1)

        def pair(p, carry):
            i0 = 2 * p
            pltpu.make_async_copy(h_hbm.at[sidx.at[i0]], rows0, gsem0).wait()
            pltpu.async_copy(rows0, acc.at[didx.at[i0]], ssem0, add=True)
            pltpu.make_async_copy(h_hbm.at[sidx.at[i0 + 1]], rows1,
                                  gsem1).wait()
            pltpu.async_copy(rows1, acc.at[didx.at[i0 + 1]], ssem1, add=True)

            @pl.when(p < PAIRS - 1)
            def _prefetch():
                pltpu.make_async_copy(rows0, acc.at[didx.at[i0]],
                                      ssem0).wait()
                pltpu.async_copy(h_hbm.at[sidx.at[i0 + 2]], rows0, gsem0)
                pltpu.make_async_copy(rows1, acc.at[didx.at[i0 + 1]],
                                      ssem1).wait()
                pltpu.async_copy(h_hbm.at[sidx.at[i0 + 3]], rows1, gsem1)

            return carry

        lax.fori_loop(0, PAIRS, pair, 0)
        pltpu.make_async_copy(rows0, acc.at[didx.at[STEPS - 3]], ssem0).wait()
        pltpu.async_copy(h_hbm.at[sidx.at[STEPS - 1]], rows0, gsem0)
        pltpu.make_async_copy(rows1, acc.at[didx.at[STEPS - 2]], ssem1).wait()
        pltpu.make_async_copy(h_hbm.at[sidx.at[STEPS - 1]], rows0,
                              gsem0).wait()
        pltpu.async_copy(rows0, acc.at[didx.at[STEPS - 1]], ssem0, add=True)
        pltpu.make_async_copy(rows0, acc.at[didx.at[STEPS - 1]], ssem0).wait()
        plsc.subcore_barrier()
        off = 0
        for n in _CHUNKS:
            pltpu.sync_copy(acc.at[pl.ds(base + off, n)],
                            out_hbm.at[c, pl.ds(base + off, n)])
            off += n

    return agg


_agg128 = _make_agg(128)
_agg64 = _make_agg(64)


GB = 2000
GRID = N // GB
_DOT = dict(preferred_element_type=jnp.float32, precision=lax.Precision.HIGHEST)


def _mm_scale_body(deg_ref, x_ref, w_ref, o_ref):
    dinv = lax.rsqrt(1.0 + deg_ref[0] + deg_ref[1])
    o_ref[...] = lax.dot_general(x_ref[...] * dinv, w_ref[...],
                                 (((1,), (0,)), ((), ())), **_DOT)


def _fused_body(agg_ref, h_ref, deg_ref, b_ref, w_ref, o_ref):
    dinv = lax.rsqrt(1.0 + deg_ref[0] + deg_ref[1])
    t = jnp.maximum(dinv * (agg_ref[0] + agg_ref[1] + h_ref[...]) + b_ref[...],
                    0.0)
    o_ref[...] = lax.dot_general(t * dinv, w_ref[...],
                                 (((1,), (0,)), ((), ())), **_DOT)


def _final_body(agg_ref, h_ref, deg_ref, b_ref, o_ref):
    dinv = lax.rsqrt(1.0 + deg_ref[0] + deg_ref[1])
    o_ref[...] = dinv * (agg_ref[0] + agg_ref[1] + h_ref[...]) + b_ref[...]


def _row_spec(d):
    return pl.BlockSpec((GB, d), lambda i: (i, 0))


_DEG_SPEC = pl.BlockSpec((NC, GB, 1), lambda i: (0, i, 0))


def _agg_spec(d):
    return pl.BlockSpec((NC, GB, d), lambda i: (0, i, 0))


def _w_spec(din, dout):
    return pl.BlockSpec((din, dout), lambda i: (0, 0))


def _b_spec(d):
    return pl.BlockSpec((1, d), lambda i: (0, 0))


_mm_scale = pl.pallas_call(
    _mm_scale_body, grid=(GRID,),
    in_specs=[_DEG_SPEC, _row_spec(128), _w_spec(128, 128)],
    out_specs=_row_spec(128),
    out_shape=jax.ShapeDtypeStruct((N, 128), jnp.float32),
)


def _make_fused(dout):
    return pl.pallas_call(
        _fused_body, grid=(GRID,),
        in_specs=[_agg_spec(128), _row_spec(128), _DEG_SPEC, _b_spec(128),
                  _w_spec(128, dout)],
        out_specs=_row_spec(dout),
        out_shape=jax.ShapeDtypeStruct((N, dout), jnp.float32),
    )


_fused128 = _make_fused(128)
_fused64 = _make_fused(64)

_final = pl.pallas_call(
    _final_body, grid=(GRID,),
    in_specs=[_agg_spec(64), _row_spec(64), _DEG_SPEC, _b_spec(64)],
    out_specs=_row_spec(64),
    out_shape=jax.ShapeDtypeStruct((N, 64), jnp.float32),
)


def kernel(x, edge_index, W1, b1, W2, b2, W3, b3):
    e4 = edge_index.astype(jnp.int32).reshape(2, NW, STEPS, B)
    ones1 = jnp.ones((B,), jnp.float32)
    zeros1 = jnp.zeros((NA,), jnp.float32)
    deg2 = _deg(e4, ones1, zeros1)[:, :, None]
    h1p = _mm_scale(deg2, x, W1)
    agg1 = _agg128(h1p, e4)
    h2p = _fused128(agg1, h1p, deg2, b1.reshape(1, -1), W2)
    agg2 = _agg128(h2p, e4)
    h3p = _fused64(agg2, h2p, deg2, b2.reshape(1, -1), W3)
    agg3 = _agg64(h3p, e4)
    return _final(agg3, h3p, deg2, b3.reshape(1, -1))


# speedup vs baseline: 1.1111x; 1.0012x over previous
import functools

import jax
import jax.numpy as jnp
from jax import lax
from jax.experimental import pallas as pl
from jax.experimental.pallas import tpu as pltpu
from jax.experimental.pallas import tpu_sc as plsc

N = 10000
E = 320000
NC = 2
NS = 16
NW = NC * NS
B = 80
STEPS = 125
PAIRS = (STEPS - 1) // 2
NA = 10112
RPT = NA // NS
_CHUNKS = [B] * (RPT // B) + ([RPT % B] if RPT % B else [])

_MESH = plsc.VectorSubcoreMesh(core_axis_name="c", subcore_axis_name="s")
_SC_PARAMS = pltpu.CompilerParams(use_tc_tiling_on_sc=False)


@functools.partial(
    pl.kernel,
    out_type=jax.ShapeDtypeStruct((NC, N), jnp.float32),
    mesh=_MESH,
    scratch_types=[
        pltpu.VMEM((B,), jnp.float32),
        pltpu.VMEM((STEPS, B), jnp.int32),
        pltpu.VMEM((640,), jnp.float32),
        pltpu.VMEM_SHARED((NA,), jnp.float32),
    ],
    compiler_params=_SC_PARAMS,
)
def _deg(e_hbm, deg_hbm, ones_v, idx_v, zbuf, acc):
    c = lax.axis_index("c")
    s = lax.axis_index("s")
    wid = c * NS + s
    one = jnp.ones((16,), jnp.float32)
    zero = jnp.zeros((16,), jnp.float32)
    for k in range(B // 16):
        ones_v[pl.ds(16 * k, 16)] = one

    @pl.when(s == 0)
    def _zero_acc():
        for k in range(40):
            zbuf[pl.ds(16 * k, 16)] = zero
        for k in range(NA // 640):
            pltpu.sync_copy(zbuf, acc.at[pl.ds(640 * k, 640)])
        rem = NA % 640
        if rem:
            pltpu.sync_copy(zbuf.at[pl.ds(0, rem)],
                            acc.at[pl.ds(NA - rem, rem)])

    pltpu.sync_copy(e_hbm.at[1, wid], idx_v)
    plsc.subcore_barrier()

    def step(i, carry):
        pltpu.sync_copy(ones_v, acc.at[idx_v.at[i]], add=True)
        return carry

    lax.fori_loop(0, STEPS, step, 0)
    plsc.subcore_barrier()

    @pl.when(s == 0)
    def _copy_out():
        pltpu.sync_copy(acc.at[pl.ds(0, N)], deg_hbm.at[c])


def _make_agg(D):
    @functools.partial(
        pl.kernel,
        out_type=jax.ShapeDtypeStruct((NC, NA, D), jnp.float32),
        mesh=_MESH,
        scratch_types=[
            pltpu.VMEM((STEPS, B), jnp.int32),
            pltpu.VMEM((STEPS, B), jnp.int32),
            pltpu.VMEM((B, D), jnp.float32),
            pltpu.VMEM((B, D), jnp.float32),
            pltpu.VMEM_SHARED((NA, D), jnp.float32),
            pltpu.SemaphoreType.DMA,
            pltpu.SemaphoreType.DMA,
            pltpu.SemaphoreType.DMA,
            pltpu.SemaphoreType.DMA,
        ],
        compiler_params=_SC_PARAMS,
    )
    def agg(h_hbm, e_hbm, out_hbm, sidx, didx, rows0, rows1, acc,
            gsem0, gsem1, ssem0, ssem1):
        c = lax.axis_index("c")
        s = lax.axis_index("s")
        wid = c * NS + s
        zero = jnp.zeros((16,), jnp.float32)

        def zrow(r, carry):
            def zcol(k, carry2):
                rows0[r, pl.ds(k * 16, 16)] = zero
                return carry2
            return lax.fori_loop(0, D // 16, zcol, carry)

        lax.fori_loop(0, B, zrow, 0)
        base = s * RPT
        off = 0
        for n in _CHUNKS:
            pltpu.sync_copy(rows0.at[pl.ds(0, n)],
                            acc.at[pl.ds(base + off, n)])
            off += n
        pltpu.sync_copy(e_hbm.at[0, wid], sidx)
        pltpu.sync_copy(e_hbm.at[1, wid], didx)
        plsc.subcore_barrier()

        pltpu.async_copy(h_hbm.at[sidx.at[0]], rows0, gsem0)
        pltpu.async_copy(h_hbm.at[sidx.at[1]], rows1, gsem1)

        def pair(p, carry):
            i0 = 2 * p
            pltpu.make_async_copy(h_hbm.at[sidx.at[i0]], rows0, gsem0).wait()
            pltpu.async_copy(rows0, acc.at[didx.at[i0]], ssem0, add=True)
            pltpu.make_async_copy(h_hbm.at[sidx.at[i0 + 1]], rows1,
                                  gsem1).wait()
            pltpu.async_copy(rows1, acc.at[didx.at[i0 + 1]], ssem1, add=True)

            @pl.when(p < PAIRS - 1)
            def _prefetch():
                pltpu.make_async_copy(rows0, acc.at[didx.at[i0]],
                                      ssem0).wait()
                pltpu.async_copy(h_hbm.at[sidx.at[i0 + 2]], rows0, gsem0)
                pltpu.make_async_copy(rows1, acc.at[didx.at[i0 + 1]],
                                      ssem1).wait()
                pltpu.async_copy(h_hbm.at[sidx.at[i0 + 3]], rows1, gsem1)

            return carry

        lax.fori_loop(0, PAIRS, pair, 0)
        pltpu.make_async_copy(rows0, acc.at[didx.at[STEPS - 3]], ssem0).wait()
        pltpu.async_copy(h_hbm.at[sidx.at[STEPS - 1]], rows0, gsem0)
        pltpu.make_async_copy(rows1, acc.at[didx.at[STEPS - 2]], ssem1).wait()
        pltpu.make_async_copy(h_hbm.at[sidx.at[STEPS - 1]], rows0,
                              gsem0).wait()
        pltpu.async_copy(rows0, acc.at[didx.at[STEPS - 1]], ssem0, add=True)
        pltpu.make_async_copy(rows0, acc.at[didx.at[STEPS - 1]], ssem0).wait()
        plsc.subcore_barrier()
        off = 0
        for n in _CHUNKS:
            pltpu.sync_copy(acc.at[pl.ds(base + off, n)],
                            out_hbm.at[c, pl.ds(base + off, n)])
            off += n

    return agg


_agg128 = _make_agg(128)
_agg64 = _make_agg(64)


GB = 2000
GRID = N // GB
_DOT = dict(preferred_element_type=jnp.float32, precision=lax.Precision.HIGHEST)


def _mm_scale_body(deg_ref, x_ref, w_ref, o_ref):
    dinv = lax.rsqrt(1.0 + deg_ref[0] + deg_ref[1])
    o_ref[...] = lax.dot_general(x_ref[...] * dinv, w_ref[...],
                                 (((1,), (0,)), ((), ())), **_DOT)


def _fused_body(agg_ref, h_ref, deg_ref, b_ref, w_ref, o_ref):
    dinv = lax.rsqrt(1.0 + deg_ref[0] + deg_ref[1])
    t = jnp.maximum(dinv * (agg_ref[0] + agg_ref[1] + h_ref[...]) + b_ref[...],
                    0.0)
    o_ref[...] = lax.dot_general(t * dinv, w_ref[...],
                                 (((1,), (0,)), ((), ())), **_DOT)


def _final_body(agg_ref, h_ref, deg_ref, b_ref, o_ref):
    dinv = lax.rsqrt(1.0 + deg_ref[0] + deg_ref[1])
    o_ref[...] = dinv * (agg_ref[0] + agg_ref[1] + h_ref[...]) + b_ref[...]


def _row_spec(d):
    return pl.BlockSpec((GB, d), lambda i: (i, 0))


_DEG_SPEC = pl.BlockSpec((NC, GB, 1), lambda i: (0, i, 0))


def _agg_spec(d):
    return pl.BlockSpec((NC, GB, d), lambda i: (0, i, 0))


def _w_spec(din, dout):
    return pl.BlockSpec((din, dout), lambda i: (0, 0))


def _b_spec(d):
    return pl.BlockSpec((1, d), lambda i: (0, 0))


_mm_scale = pl.pallas_call(
    _mm_scale_body, grid=(GRID,),
    in_specs=[_DEG_SPEC, _row_spec(128), _w_spec(128, 128)],
    out_specs=_row_spec(128),
    out_shape=jax.ShapeDtypeStruct((N, 128), jnp.float32),
)


def _make_fused(dout):
    return pl.pallas_call(
        _fused_body, grid=(GRID,),
        in_specs=[_agg_spec(128), _row_spec(128), _DEG_SPEC, _b_spec(128),
                  _w_spec(128, dout)],
        out_specs=_row_spec(dout),
        out_shape=jax.ShapeDtypeStruct((N, dout), jnp.float32),
    )


_fused128 = _make_fused(128)
_fused64 = _make_fused(64)

_final = pl.pallas_call(
    _final_body, grid=(GRID,),
    in_specs=[_agg_spec(64), _row_spec(64), _DEG_SPEC, _b_spec(64)],
    out_specs=_row_spec(64),
    out_shape=jax.ShapeDtypeStruct((N, 64), jnp.float32),
)


def kernel(x, edge_index, W1, b1, W2, b2, W3, b3):
    e4 = edge_index.astype(jnp.int32).reshape(2, NW, STEPS, B)
    deg2 = _deg(e4)[:, :, None]
    h1p = _mm_scale(deg2, x, W1)
    agg1 = _agg128(h1p, e4)
    h2p = _fused128(agg1, h1p, deg2, b1.reshape(1, -1), W2)
    agg2 = _agg128(h2p, e4)
    h3p = _fused64(agg2, h2p, deg2, b2.reshape(1, -1), W3)
    agg3 = _agg64(h3p, e4)
    return _final(agg3, h3p, deg2, b3.reshape(1, -1))
